# baseline (speedup 1.0000x reference)
import jax
import jax.numpy as jnp
from jax.experimental import pallas as pl

N = 10000
NCLASS = 32


def _hamilton(k):
    r, i, j, q = jnp.split(k, 4, axis=1)
    r2 = jnp.concatenate([r, -i, -j, -q], axis=0)
    i2 = jnp.concatenate([i, r, -q, j], axis=0)
    j2 = jnp.concatenate([j, q, r, -i], axis=0)
    k2 = jnp.concatenate([q, -j, i, r], axis=0)
    return jnp.concatenate([r2, i2, j2, k2], axis=1)


def _spmm(edge_index, edge_weight, support):
    row = edge_index[0]
    col = edge_index[1]
    msgs = support[col] * edge_weight[:, None]
    return jax.ops.segment_sum(msgs, row, num_segments=N)


def _lsm_body(z_ref, o_ref):
    z = z_ref[...]
    m = jnp.max(z, axis=1, keepdims=True)
    e = jnp.exp(z - m)
    s = jnp.sum(e, axis=1, keepdims=True)
    o_ref[...] = z - m - jnp.log(s)


def _log_softmax(z):
    return pl.pallas_call(
        _lsm_body,
        out_shape=jax.ShapeDtypeStruct(z.shape, z.dtype),
        grid=(10,),
        in_specs=[pl.BlockSpec((N // 10, NCLASS), lambda i: (i, 0))],
        out_specs=pl.BlockSpec((N // 10, NCLASS), lambda i: (i, 0)),
    )(z)


def kernel(x, edge_index, edge_weight, now_epoch, W0, W1, Wm1, Ws2, Ws3, Ws4, Ws5, Wout, gc_W1, gc_b1, gc_W12, gc_b12, gc_W2, gc_b2):
    y0 = jnp.tanh(_spmm(edge_index, edge_weight, x @ _hamilton(W0)))
    h = jnp.tanh(_spmm(edge_index, edge_weight, x @ _hamilton(W1)))
    x1, x2, x3, x4 = jnp.split(h, 4, axis=1)
    y1 = jnp.tanh(_spmm(edge_index, edge_weight, x1 @ _hamilton(Ws2)))
    y12 = _spmm(edge_index, edge_weight, y1 @ Wm1)
    x2 = y12 * x2
    y2 = jnp.tanh(_spmm(edge_index, edge_weight, x2 @ _hamilton(Ws3)))
    y22 = _spmm(edge_index, edge_weight, y2 @ Wm1)
    x3 = y22 * x3
    y3 = jnp.tanh(_spmm(edge_index, edge_weight, x3 @ _hamilton(Ws4)))
    y32 = _spmm(edge_index, edge_weight, y3 @ Wm1)
    x4 = y32 * x4
    y4 = jnp.tanh(_spmm(edge_index, edge_weight, x4 @ _hamilton(Ws5)))
    y = jnp.concatenate([y1, y2, y3, y4], axis=1)
    m = jnp.mean(y, axis=0)
    a = m @ gc_W1 + gc_b1
    t = m @ gc_W12 + gc_b12
    v = jax.nn.relu(jnp.concatenate([a * jnp.cos(t), a * jnp.sin(t)]))
    yc = jax.nn.sigmoid(v @ gc_W2 + gc_b2)
    y = y * yc + y0
    out = _spmm(edge_index, edge_weight, y @ Wout)
    return _log_softmax(out)


# speedup vs baseline: 3.2088x; 3.2088x over previous
import functools

import jax
import jax.numpy as jnp
from jax import lax
from jax.experimental import pallas as pl
from jax.experimental.pallas import tpu as pltpu
from jax.experimental.pallas import tpu_sc as plsc

N = 10000
E = 160000
NFEAT = 256
NHID = 512
NCLASS = 32

NC = 2
NS = 16
NW = NC * NS
EB = 128
NBLK = 40
EPT = EB * NBLK
EPAD = EPT * NW
NPAD = 10240
RPT = NPAD // NS
ZR = 128


def _hamilton(k):
    r, i, j, q = jnp.split(k, 4, axis=1)
    r2 = jnp.concatenate([r, -i, -j, -q], axis=0)
    i2 = jnp.concatenate([i, r, -q, j], axis=0)
    j2 = jnp.concatenate([j, q, r, -i], axis=0)
    k2 = jnp.concatenate([q, -j, i, r], axis=0)
    return jnp.concatenate([r2, i2, j2, k2], axis=1)


_GDN = lax.GatherDimensionNumbers(
    offset_dims=(), collapsed_slice_dims=(0,), start_index_map=(0,))


def _bcast_lane(vec, lane):
    idx = jnp.full((16, 1), lane, jnp.int32)
    return lax.gather(vec, idx, _GDN, (1,),
                      mode=lax.GatherScatterMode.PROMISE_IN_BOUNDS)


def _spmm_body(F, src_hbm, col_hbm, row_hbm, w_hbm, out_hbm,
               col_v, row_v, w_v, rows_v, zb_v, acc_sh, sem):
    c = lax.axis_index("c")
    s = lax.axis_index("s")
    wid = s * NC + c
    CH = F // 16

    pltpu.sync_copy(col_hbm.at[wid], col_v)
    pltpu.sync_copy(row_hbm.at[wid], row_v)
    pltpu.sync_copy(w_hbm.at[wid], w_v)

    zv = jnp.zeros((16,), jnp.float32)

    def zrow(r, _):
        for kk in range(CH):
            zb_v[r, pl.ds(kk * 16, 16)] = zv
        return 0

    lax.fori_loop(0, ZR, zrow, 0)
    for i in range(RPT // ZR):
        pltpu.sync_copy(zb_v, acc_sh.at[pl.ds(s * RPT + i * ZR, ZR)])
    plsc.subcore_barrier()

    def blk(j, _):
        pltpu.async_copy(src_hbm.at[col_v.at[j]], rows_v, sem).wait()

        def grp(g, _):
            wblk = w_v[pl.ds(j * EB + g * 16, 16)]
            for e16 in range(16):
                wv = _bcast_lane(wblk, e16)
                e = g * 16 + e16
                for kk in range(CH):
                    sl = pl.ds(kk * 16, 16)
                    rows_v[e, sl] = rows_v[e, sl] * wv
            return 0

        lax.fori_loop(0, EB // 16, grp, 0)
        pltpu.sync_copy(rows_v, acc_sh.at[row_v.at[j]], add=True)
        return 0

    lax.fori_loop(0, NBLK, blk, 0)
    plsc.subcore_barrier()

    for i in range(RPT // ZR):
        pltpu.sync_copy(acc_sh.at[pl.ds(s * RPT + i * ZR, ZR)], zb_v)
        pltpu.sync_copy(zb_v, out_hbm.at[c].at[pl.ds(s * RPT + i * ZR, ZR)])


def _make_spmm(F):
    mesh = plsc.VectorSubcoreMesh(core_axis_name="c", subcore_axis_name="s")
    return pl.kernel(
        functools.partial(_spmm_body, F),
        mesh=mesh,
        out_type=jax.ShapeDtypeStruct((NC, NPAD, F), jnp.float32),
        scratch_types=[
            pltpu.VMEM((NBLK, EB), jnp.int32),
            pltpu.VMEM((NBLK, EB), jnp.int32),
            pltpu.VMEM((EPT,), jnp.float32),
            pltpu.VMEM((EB, F), jnp.float32),
            pltpu.VMEM((ZR, F), jnp.float32),
            pltpu.VMEM_SHARED((NPAD, F), jnp.float32),
            pltpu.SemaphoreType.DMA,
        ],
        name=f"spmm_sc_f{F}",
    )


_spmm128 = _make_spmm(128)


_RB = 1000


def _big_body(p_ref, q_ref, w_ref, o_ref):
    a = p_ref[0] + p_ref[1]
    b = q_ref[0] + q_ref[1]
    xb = jnp.concatenate([a, b], axis=1)
    o_ref[...] = jnp.tanh(jnp.dot(xb, w_ref[...],
                                  preferred_element_type=jnp.float32))


def _big_mm(p, q, w):
    return pl.pallas_call(
        _big_body,
        out_shape=jax.ShapeDtypeStruct((N, 2 * NHID), jnp.float32),
        grid=(N // _RB,),
        in_specs=[
            pl.BlockSpec((NC, _RB, 128), lambda i: (0, i, 0)),
            pl.BlockSpec((NC, _RB, 128), lambda i: (0, i, 0)),
            pl.BlockSpec((NFEAT, 2 * NHID), lambda i: (0, 0)),
        ],
        out_specs=pl.BlockSpec((_RB, 2 * NHID), lambda i: (i, 0)),
    )(p, q, w)


def _mm_tanh_body(p_ref, w_ref, o_ref):
    a = p_ref[0] + p_ref[1]
    o_ref[...] = jnp.tanh(jnp.dot(a, w_ref[...],
                                  preferred_element_type=jnp.float32))


def _mm_tanh(p, w):
    return pl.pallas_call(
        _mm_tanh_body,
        out_shape=jax.ShapeDtypeStruct((N, 128), jnp.float32),
        grid=(N // _RB,),
        in_specs=[
            pl.BlockSpec((NC, _RB, 128), lambda i: (0, i, 0)),
            pl.BlockSpec((128, 128), lambda i: (0, 0)),
        ],
        out_specs=pl.BlockSpec((_RB, 128), lambda i: (i, 0)),
    )(p, w)


def _mm_mul_body(p_ref, w_ref, z_ref, o_ref):
    a = p_ref[0] + p_ref[1]
    g = jnp.dot(a, w_ref[...], preferred_element_type=jnp.float32)
    o_ref[...] = g * z_ref[...]


def _mm_mul(p, w, z):
    return pl.pallas_call(
        _mm_mul_body,
        out_shape=jax.ShapeDtypeStruct((N, 128), jnp.float32),
        grid=(N // _RB,),
        in_specs=[
            pl.BlockSpec((NC, _RB, 128), lambda i: (0, i, 0)),
            pl.BlockSpec((128, 128), lambda i: (0, 0)),
            pl.BlockSpec((_RB, 128), lambda i: (i, 0)),
        ],
        out_specs=pl.BlockSpec((_RB, 128), lambda i: (i, 0)),
    )(p, w, z)


def _mean_body(y_ref, o_ref):
    i = pl.program_id(0)

    @pl.when(i == 0)
    def _():
        o_ref[...] = jnp.zeros_like(o_ref)

    o_ref[...] += jnp.sum(y_ref[...], axis=0, keepdims=True) * (1.0 / N)


def _col_mean(y):
    return pl.pallas_call(
        _mean_body,
        out_shape=jax.ShapeDtypeStruct((1, NHID), jnp.float32),
        grid=(N // _RB,),
        in_specs=[pl.BlockSpec((_RB, NHID), lambda i: (i, 0))],
        out_specs=pl.BlockSpec((1, NHID), lambda i: (0, 0)),
    )(y)


def _gate_body(m_ref, w1_ref, b1_ref, w12_ref, b12_ref, w2a_ref, w2b_ref,
               b2_ref, o_ref):
    m = m_ref[...]
    a = jnp.dot(m, w1_ref[...], preferred_element_type=jnp.float32) + b1_ref[...]
    t = jnp.dot(m, w12_ref[...], preferred_element_type=jnp.float32) + b12_ref[...]
    va = jax.nn.relu(a * jnp.cos(t))
    vb = jax.nn.relu(a * jnp.sin(t))
    v = (jnp.dot(va, w2a_ref[...], preferred_element_type=jnp.float32)
         + jnp.dot(vb, w2b_ref[...], preferred_element_type=jnp.float32)
         + b2_ref[...])
    o_ref[...] = jax.nn.sigmoid(v)


def _gate(m, w1p, b1p, w12p, b12p, w2a, w2b, b2):
    full = lambda shp: pl.BlockSpec(shp, lambda: tuple(0 for _ in shp))
    return pl.pallas_call(
        _gate_body,
        out_shape=jax.ShapeDtypeStruct((1, NHID), jnp.float32),
        in_specs=[full((1, NHID)), full((NHID, 128)), full((1, 128)),
                  full((NHID, 128)), full((1, 128)), full((128, NHID)),
                  full((128, NHID)), full((1, NHID))],
        out_specs=full((1, NHID)),
    )(m, w1p, b1p, w12p, b12p, w2a, w2b, b2)


def _out_body(y_ref, yc_ref, y0_ref, w_ref, o_ref):
    yg = y_ref[...] * yc_ref[...] + y0_ref[...]
    o_ref[...] = jnp.dot(yg, w_ref[...], preferred_element_type=jnp.float32)


def _out_mm(y, yc, y0, w):
    return pl.pallas_call(
        _out_body,
        out_shape=jax.ShapeDtypeStruct((N, 128), jnp.float32),
        grid=(N // _RB,),
        in_specs=[
            pl.BlockSpec((_RB, NHID), lambda i: (i, 0)),
            pl.BlockSpec((1, NHID), lambda i: (0, 0)),
            pl.BlockSpec((_RB, NHID), lambda i: (i, 0)),
            pl.BlockSpec((NHID, 128), lambda i: (0, 0)),
        ],
        out_specs=pl.BlockSpec((_RB, 128), lambda i: (i, 0)),
    )(y, yc, y0, w)


def _lsm_body(p_ref, o_ref):
    z = (p_ref[0] + p_ref[1])[:, :NCLASS]
    mx = jnp.max(z, axis=1, keepdims=True)
    ez = jnp.exp(z - mx)
    sz = jnp.sum(ez, axis=1, keepdims=True)
    o_ref[...] = z - mx - jnp.log(sz)


def _lsm(p):
    return pl.pallas_call(
        _lsm_body,
        out_shape=jax.ShapeDtypeStruct((N, NCLASS), jnp.float32),
        grid=(N // _RB,),
        in_specs=[pl.BlockSpec((NC, _RB, 128), lambda i: (0, i, 0))],
        out_specs=pl.BlockSpec((_RB, NCLASS), lambda i: (i, 0)),
    )(p)


def kernel(x, edge_index, edge_weight, now_epoch, W0, W1, Wm1, Ws2, Ws3, Ws4,
           Ws5, Wout, gc_W1, gc_b1, gc_W12, gc_b12, gc_W2, gc_b2):
    pad = EPAD - E
    col = jnp.pad(edge_index[1], (0, pad)).reshape(NW, NBLK, EB)
    row = jnp.pad(edge_index[0], (0, pad)).reshape(NW, NBLK, EB)
    w = jnp.pad(edge_weight, (0, pad)).reshape(NW, EPT)

    spmm128 = lambda src: _spmm128(src, col, row, w)

    wbig = jnp.concatenate([_hamilton(W0), _hamilton(W1)], axis=1)
    hs2, hs3, hs4, hs5 = (_hamilton(Ws) for Ws in (Ws2, Ws3, Ws4, Ws5))
    padc = lambda a: jnp.pad(a, ((0, 0), (0, 128 - a.shape[1])))
    w1p = padc(gc_W1)
    w12p = padc(gc_W12)
    b1p = padc(gc_b1[None, :])
    b12p = padc(gc_b12[None, :])
    w2a = jnp.pad(gc_W2[:7], ((0, 121), (0, 0)))
    w2b = jnp.pad(gc_W2[7:], ((0, 121), (0, 0)))
    b2 = gc_b2[None, :]
    wout_p = jnp.pad(Wout, ((0, 0), (0, 128 - NCLASS)))

    axa = spmm128(x[:, :128])
    axb = spmm128(x[:, 128:])
    yh = _big_mm(axa, axb, wbig)
    y0 = yh[:, :NHID]
    x1 = yh[:, NHID:NHID + 128]
    x2 = yh[:, NHID + 128:NHID + 256]
    x3 = yh[:, NHID + 256:NHID + 384]
    x4 = yh[:, NHID + 384:]

    y1 = _mm_tanh(spmm128(x1), hs2)
    x2m = _mm_mul(spmm128(y1), Wm1, x2)
    y2 = _mm_tanh(spmm128(x2m), hs3)
    x3m = _mm_mul(spmm128(y2), Wm1, x3)
    y3 = _mm_tanh(spmm128(x3m), hs4)
    x4m = _mm_mul(spmm128(y3), Wm1, x4)
    y4 = _mm_tanh(spmm128(x4m), hs5)

    y = jnp.concatenate([y1, y2, y3, y4], axis=1)
    m = _col_mean(y)
    yc = _gate(m, w1p, b1p, w12p, b12p, w2a, w2b, b2)
    z = _out_mm(y, yc, y0, wout_p)
    return _lsm(spmm128(z))


# speedup vs baseline: 3.9097x; 1.2184x over previous
import functools

import jax
import jax.numpy as jnp
from jax import lax
from jax.experimental import pallas as pl
from jax.experimental.pallas import tpu as pltpu
from jax.experimental.pallas import tpu_sc as plsc

N = 10000
E = 160000
NFEAT = 256
NHID = 512
NCLASS = 32

NC = 2
NS = 16
NW = NC * NS
EB = 128
NBLK = 40
EPT = EB * NBLK
EPAD = EPT * NW
NPAD = 10240
RPT = NPAD // NS
ZR = 128


def _hamilton(k):
    r, i, j, q = jnp.split(k, 4, axis=1)
    r2 = jnp.concatenate([r, -i, -j, -q], axis=0)
    i2 = jnp.concatenate([i, r, -q, j], axis=0)
    j2 = jnp.concatenate([j, q, r, -i], axis=0)
    k2 = jnp.concatenate([q, -j, i, r], axis=0)
    return jnp.concatenate([r2, i2, j2, k2], axis=1)


_GDN = lax.GatherDimensionNumbers(
    offset_dims=(), collapsed_slice_dims=(0,), start_index_map=(0,))


def _bcast_lane(vec, lane):
    idx = jnp.full((16, 1), lane, jnp.int32)
    return lax.gather(vec, idx, _GDN, (1,),
                      mode=lax.GatherScatterMode.PROMISE_IN_BOUNDS)


_NBUF = 2


def _spmm_body(F, SCH, src_hbm, col_hbm, row_hbm, w_hbm, out_hbm,
               col_v, row_v, w_v, rows_v, acc_sh, g0, g1):
    c = lax.axis_index("c")
    s = lax.axis_index("s")
    wid = s * NC + c
    CH = F // 16
    gsem = [g0, g1]

    pltpu.sync_copy(col_hbm.at[wid], col_v)
    pltpu.sync_copy(row_hbm.at[wid], row_v)
    pltpu.sync_copy(w_hbm.at[wid], w_v)

    zb_v = rows_v.at[0]
    zv = jnp.zeros((16,), jnp.float32)

    def zrow(r, _):
        for kk in range(CH):
            zb_v[r, pl.ds(kk * 16, 16)] = zv
        return 0

    lax.fori_loop(0, ZR, zrow, 0)
    for i in range(RPT // ZR):
        pltpu.sync_copy(zb_v, acc_sh.at[pl.ds(s * RPT + i * ZR, ZR)])
    pltpu.async_copy(src_hbm.at[col_v.at[0]], rows_v.at[0], gsem[0])
    pltpu.async_copy(src_hbm.at[col_v.at[1]], rows_v.at[1], gsem[1])
    plsc.subcore_barrier()

    def pair(q, _):
        for b in range(_NBUF):
            j = q * _NBUF + b
            rv = rows_v.at[b]
            pltpu.make_async_copy(src_hbm.at[col_v.at[j]], rv, gsem[b]).wait()

            def grp(g, _):
                wblk = w_v[pl.ds(j * EB + g * 16, 16)]
                for e16 in range(16):
                    wv = _bcast_lane(wblk, e16)
                    e = g * 16 + e16
                    for kk in range(SCH):
                        sl = pl.ds(kk * 16, 16)
                        rv[e, sl] = rv[e, sl] * wv
                return 0

            lax.fori_loop(0, EB // 16, grp, 0)
            pltpu.sync_copy(rv, acc_sh.at[row_v.at[j]], add=True)

            @pl.when(j + _NBUF < NBLK)
            def _():
                pltpu.async_copy(
                    src_hbm.at[col_v.at[j + _NBUF]], rv, gsem[b])
        return 0

    lax.fori_loop(0, NBLK // _NBUF, pair, 0)
    plsc.subcore_barrier()

    for i in range(RPT // ZR):
        pltpu.sync_copy(acc_sh.at[pl.ds(s * RPT + i * ZR, ZR)], zb_v)
        pltpu.sync_copy(zb_v, out_hbm.at[c].at[pl.ds(s * RPT + i * ZR, ZR)])


def _make_spmm(F, sch):
    mesh = plsc.VectorSubcoreMesh(core_axis_name="c", subcore_axis_name="s")
    return pl.kernel(
        functools.partial(_spmm_body, F, sch),
        mesh=mesh,
        out_type=jax.ShapeDtypeStruct((NC, NPAD, F), jnp.float32),
        scratch_types=[
            pltpu.VMEM((NBLK, EB), jnp.int32),
            pltpu.VMEM((NBLK, EB), jnp.int32),
            pltpu.VMEM((EPT,), jnp.float32),
            pltpu.VMEM((_NBUF, EB, F), jnp.float32),
            pltpu.VMEM_SHARED((NPAD, F), jnp.float32),
        ] + [pltpu.SemaphoreType.DMA] * 2,
        name=f"spmm_sc_f{F}s{sch}",
    )


_spmm128 = _make_spmm(128, 8)
_spmm128c2 = _make_spmm(128, 2)


_RB = 1000


def _big_body(p_ref, q_ref, w_ref, o_ref):
    a = p_ref[0] + p_ref[1]
    b = q_ref[0] + q_ref[1]
    xb = jnp.concatenate([a, b], axis=1)
    o_ref[...] = jnp.tanh(jnp.dot(xb, w_ref[...],
                                  preferred_element_type=jnp.float32))


def _big_mm(p, q, w):
    return pl.pallas_call(
        _big_body,
        out_shape=jax.ShapeDtypeStruct((N, 2 * NHID), jnp.float32),
        grid=(N // _RB,),
        in_specs=[
            pl.BlockSpec((NC, _RB, 128), lambda i: (0, i, 0)),
            pl.BlockSpec((NC, _RB, 128), lambda i: (0, i, 0)),
            pl.BlockSpec((NFEAT, 2 * NHID), lambda i: (0, 0)),
        ],
        out_specs=pl.BlockSpec((_RB, 2 * NHID), lambda i: (i, 0)),
    )(p, q, w)


def _mm_tanh_body(p_ref, w_ref, o_ref):
    a = p_ref[0] + p_ref[1]
    o_ref[...] = jnp.tanh(jnp.dot(a, w_ref[...],
                                  preferred_element_type=jnp.float32))


def _mm_tanh(p, w):
    return pl.pallas_call(
        _mm_tanh_body,
        out_shape=jax.ShapeDtypeStruct((N, 128), jnp.float32),
        grid=(N // _RB,),
        in_specs=[
            pl.BlockSpec((NC, _RB, 128), lambda i: (0, i, 0)),
            pl.BlockSpec((128, 128), lambda i: (0, 0)),
        ],
        out_specs=pl.BlockSpec((_RB, 128), lambda i: (i, 0)),
    )(p, w)


def _mm_mul_body(p_ref, w_ref, z_ref, o_ref):
    a = p_ref[0] + p_ref[1]
    g = jnp.dot(a, w_ref[...], preferred_element_type=jnp.float32)
    o_ref[...] = g * z_ref[...]


def _mm_mul(p, w, z):
    return pl.pallas_call(
        _mm_mul_body,
        out_shape=jax.ShapeDtypeStruct((N, 128), jnp.float32),
        grid=(N // _RB,),
        in_specs=[
            pl.BlockSpec((NC, _RB, 128), lambda i: (0, i, 0)),
            pl.BlockSpec((128, 128), lambda i: (0, 0)),
            pl.BlockSpec((_RB, 128), lambda i: (i, 0)),
        ],
        out_specs=pl.BlockSpec((_RB, 128), lambda i: (i, 0)),
    )(p, w, z)


def _mean_body(y_ref, o_ref):
    i = pl.program_id(0)

    @pl.when(i == 0)
    def _():
        o_ref[...] = jnp.zeros_like(o_ref)

    o_ref[...] += jnp.sum(y_ref[...], axis=0, keepdims=True) * (1.0 / N)


def _col_mean(y):
    return pl.pallas_call(
        _mean_body,
        out_shape=jax.ShapeDtypeStruct((1, NHID), jnp.float32),
        grid=(N // _RB,),
        in_specs=[pl.BlockSpec((_RB, NHID), lambda i: (i, 0))],
        out_specs=pl.BlockSpec((1, NHID), lambda i: (0, 0)),
    )(y)


def _gate_body(m_ref, w1_ref, b1_ref, w12_ref, b12_ref, w2a_ref, w2b_ref,
               b2_ref, o_ref):
    m = m_ref[...]
    a = jnp.dot(m, w1_ref[...], preferred_element_type=jnp.float32) + b1_ref[...]
    t = jnp.dot(m, w12_ref[...], preferred_element_type=jnp.float32) + b12_ref[...]
    va = jax.nn.relu(a * jnp.cos(t))
    vb = jax.nn.relu(a * jnp.sin(t))
    v = (jnp.dot(va, w2a_ref[...], preferred_element_type=jnp.float32)
         + jnp.dot(vb, w2b_ref[...], preferred_element_type=jnp.float32)
         + b2_ref[...])
    o_ref[...] = jax.nn.sigmoid(v)


def _gate(m, w1p, b1p, w12p, b12p, w2a, w2b, b2):
    full = lambda shp: pl.BlockSpec(shp, lambda: tuple(0 for _ in shp))
    return pl.pallas_call(
        _gate_body,
        out_shape=jax.ShapeDtypeStruct((1, NHID), jnp.float32),
        in_specs=[full((1, NHID)), full((NHID, 128)), full((1, 128)),
                  full((NHID, 128)), full((1, 128)), full((128, NHID)),
                  full((128, NHID)), full((1, NHID))],
        out_specs=full((1, NHID)),
    )(m, w1p, b1p, w12p, b12p, w2a, w2b, b2)


def _out_body(y_ref, yc_ref, y0_ref, w_ref, o_ref):
    yg = y_ref[...] * yc_ref[...] + y0_ref[...]
    o_ref[...] = jnp.dot(yg, w_ref[...], preferred_element_type=jnp.float32)


def _out_mm(y, yc, y0, w):
    return pl.pallas_call(
        _out_body,
        out_shape=jax.ShapeDtypeStruct((N, 128), jnp.float32),
        grid=(N // _RB,),
        in_specs=[
            pl.BlockSpec((_RB, NHID), lambda i: (i, 0)),
            pl.BlockSpec((1, NHID), lambda i: (0, 0)),
            pl.BlockSpec((_RB, NHID), lambda i: (i, 0)),
            pl.BlockSpec((NHID, 128), lambda i: (0, 0)),
        ],
        out_specs=pl.BlockSpec((_RB, 128), lambda i: (i, 0)),
    )(y, yc, y0, w)


def _lsm_body(p_ref, o_ref):
    z = (p_ref[0] + p_ref[1])[:, :NCLASS]
    mx = jnp.max(z, axis=1, keepdims=True)
    ez = jnp.exp(z - mx)
    sz = jnp.sum(ez, axis=1, keepdims=True)
    o_ref[...] = z - mx - jnp.log(sz)


def _lsm(p):
    return pl.pallas_call(
        _lsm_body,
        out_shape=jax.ShapeDtypeStruct((N, NCLASS), jnp.float32),
        grid=(N // _RB,),
        in_specs=[pl.BlockSpec((NC, _RB, 128), lambda i: (0, i, 0))],
        out_specs=pl.BlockSpec((_RB, NCLASS), lambda i: (i, 0)),
    )(p)


def kernel(x, edge_index, edge_weight, now_epoch, W0, W1, Wm1, Ws2, Ws3, Ws4,
           Ws5, Wout, gc_W1, gc_b1, gc_W12, gc_b12, gc_W2, gc_b2):
    pad = EPAD - E
    col = jnp.pad(edge_index[1], (0, pad)).reshape(NW, NBLK, EB)
    row = jnp.pad(edge_index[0], (0, pad)).reshape(NW, NBLK, EB)
    w = jnp.pad(edge_weight, (0, pad)).reshape(NW, EPT)

    spmm128 = lambda src: _spmm128(src, col, row, w)

    wbig = jnp.concatenate([_hamilton(W0), _hamilton(W1)], axis=1)
    hs2, hs3, hs4, hs5 = (_hamilton(Ws) for Ws in (Ws2, Ws3, Ws4, Ws5))
    padc = lambda a: jnp.pad(a, ((0, 0), (0, 128 - a.shape[1])))
    w1p = padc(gc_W1)
    w12p = padc(gc_W12)
    b1p = padc(gc_b1[None, :])
    b12p = padc(gc_b12[None, :])
    w2a = jnp.pad(gc_W2[:7], ((0, 121), (0, 0)))
    w2b = jnp.pad(gc_W2[7:], ((0, 121), (0, 0)))
    b2 = gc_b2[None, :]
    wout_p = jnp.pad(Wout, ((0, 0), (0, 128 - NCLASS)))

    axa = spmm128(x[:, :128])
    axb = spmm128(x[:, 128:])
    yh = _big_mm(axa, axb, wbig)
    y0 = yh[:, :NHID]
    x1 = yh[:, NHID:NHID + 128]
    x2 = yh[:, NHID + 128:NHID + 256]
    x3 = yh[:, NHID + 256:NHID + 384]
    x4 = yh[:, NHID + 384:]

    y1 = _mm_tanh(spmm128(x1), hs2)
    x2m = _mm_mul(spmm128(y1), Wm1, x2)
    y2 = _mm_tanh(spmm128(x2m), hs3)
    x3m = _mm_mul(spmm128(y2), Wm1, x3)
    y3 = _mm_tanh(spmm128(x3m), hs4)
    x4m = _mm_mul(spmm128(y3), Wm1, x4)
    y4 = _mm_tanh(spmm128(x4m), hs5)

    y = jnp.concatenate([y1, y2, y3, y4], axis=1)
    m = _col_mean(y)
    yc = _gate(m, w1p, b1p, w12p, b12p, w2a, w2b, b2)
    z = _out_mm(y, yc, y0, wout_p)
    return _lsm(_spmm128c2(z, col, row, w))
